# baseline (speedup 1.0000x reference)
import functools

import jax
import jax.numpy as jnp
from jax import lax
from jax.experimental import pallas as pl


def _sqdist(a, b):
    aa = jnp.sum(a * a, axis=-1)[:, :, None]
    bb = jnp.sum(b * b, axis=-1)[:, None, :]
    ab = jnp.einsum('bmc,bnc->bmn', a, b)
    return jnp.maximum(aa + bb - 2.0 * ab, 0.0)


def _gather(pts, idx):
    return jax.vmap(lambda p, i: p[i])(pts, idx)


def _knn(K, xyz, new_xyz):
    d2 = _sqdist(new_xyz, xyz)
    _, idx = jax.lax.top_k(-d2, K)
    return idx


def _kde_ball(pts, radius, sigma, n_kde):
    d2 = _sqdist(pts, pts)
    negv, _ = jax.lax.top_k(-d2, n_kde)
    d2k = -negv
    mask = d2k <= radius * radius
    cnt = jnp.maximum(jnp.sum(mask, axis=-1), 1).astype(jnp.float32)
    R = jnp.sqrt(sigma)
    logpdf = -0.5 * d2k / sigma - 3.0 * jnp.log(R) - 1.5 * jnp.log(2.0 * jnp.pi)
    pdf = jnp.where(mask, jnp.exp(logpdf), 0.0)
    density = jnp.sum(pdf, axis=-1) / cnt
    return density[..., None]


def _fps_kernel(x_ref, o_ref, *, npoint, n):
    pts = x_ref[0]
    xr = pts[0:1, :]
    yr = pts[1:2, :]
    zr = pts[2:3, :]
    iota_n = lax.broadcasted_iota(jnp.int32, (1, n), 1)
    iota_p = lax.broadcasted_iota(jnp.int32, (1, npoint), 1)

    def body(i, carry):
        dists, last, idxs = carry
        sel = iota_n == last
        px = jnp.sum(jnp.where(sel, xr, 0.0))
        py = jnp.sum(jnp.where(sel, yr, 0.0))
        pz = jnp.sum(jnp.where(sel, zr, 0.0))
        d = (xr - px) ** 2 + (yr - py) ** 2 + (zr - pz) ** 2
        dists = jnp.minimum(dists, d)
        m = jnp.max(dists)
        nxt = jnp.min(jnp.where(dists == m, iota_n, n))
        idxs = jnp.where(iota_p == i + 1, nxt, idxs)
        return dists, nxt, idxs

    dists0 = jnp.full((1, n), 1e10, jnp.float32)
    idxs0 = jnp.zeros((1, npoint), jnp.int32)
    _, _, idxs = lax.fori_loop(0, npoint - 1, body,
                               (dists0, jnp.int32(0), idxs0))
    o_ref[...] = idxs[None]


def _fps(points, npoint):
    B, N, _ = points.shape
    xt = jnp.transpose(points, (0, 2, 1))
    out = pl.pallas_call(
        functools.partial(_fps_kernel, npoint=npoint, n=N),
        grid=(B,),
        in_specs=[pl.BlockSpec((1, 3, N), lambda i: (i, 0, 0))],
        out_specs=pl.BlockSpec((1, 1, npoint), lambda i: (i, 0, 0)),
        out_shape=jax.ShapeDtypeStruct((B, 1, npoint), jnp.int32),
    )(xt)
    return out[:, 0]


def _sa_core(x_ref, d_ref, w1_ref, b1_ref, w2_ref, b2_ref,
             wn_ref, bn_ref, wd1_ref, bd1_ref, wd2_ref, bd2_ref):
    Gt, K, Cin = x_ref.shape
    X = x_ref[...].reshape(Gt * K, Cin)
    h = jnp.maximum(
        jnp.dot(X, w1_ref[...], preferred_element_type=jnp.float32)
        + b1_ref[...], 0.0)
    h = jnp.maximum(
        jnp.dot(h, w2_ref[...], preferred_element_type=jnp.float32)
        + b2_ref[...], 0.0)
    wn = wn_ref[...]
    w = (X[:, 0:1] * wn[0:1, :] + X[:, 1:2] * wn[1:2, :]
         + X[:, 2:3] * wn[2:3, :]) + bn_ref[...]
    w = jnp.maximum(w, 0.0)
    dv = d_ref[...].reshape(Gt * K, 1)
    ds = jnp.maximum(dv * wd1_ref[...] + bd1_ref[...], 0.0)
    ds = jax.nn.sigmoid(jnp.sum(ds * wd2_ref[...], axis=1, keepdims=True)
                        + bd2_ref[...])
    h = h * ds
    C = h.shape[1]
    hb = h.reshape(Gt, K, C)
    wb = w.reshape(Gt, K, 32)
    o = lax.dot_general(hb, wb, (((1,), (1,)), ((0,), (0,))),
                        preferred_element_type=jnp.float32)
    return o.reshape(Gt, C * 32)


def _sa_mlp_final_kernel(x_ref, d_ref, w1_ref, b1_ref, w2_ref, b2_ref,
                         wn_ref, bn_ref, wd1_ref, bd1_ref, wd2_ref, bd2_ref,
                         wf_ref, bf_ref, o_ref):
    o = _sa_core(x_ref, d_ref, w1_ref, b1_ref, w2_ref, b2_ref,
                 wn_ref, bn_ref, wd1_ref, bd1_ref, wd2_ref, bd2_ref)
    o_ref[...] = jnp.maximum(
        jnp.dot(o, wf_ref[...], preferred_element_type=jnp.float32)
        + bf_ref[...], 0.0)


def _sa_mlp_nofinal_kernel(x_ref, d_ref, w1_ref, b1_ref, w2_ref, b2_ref,
                           wn_ref, bn_ref, wd1_ref, bd1_ref, wd2_ref,
                           bd2_ref, o_ref):
    o_ref[...] = _sa_core(x_ref, d_ref, w1_ref, b1_ref, w2_ref, b2_ref,
                          wn_ref, bn_ref, wd1_ref, bd1_ref, wd2_ref, bd2_ref)


def _full(shape):
    nd = len(shape)
    return pl.BlockSpec(shape, lambda *_: (0,) * nd)


def _row(x):
    return x.reshape(1, -1)


def _sa_mlp(feat, dsc, p, gt, fuse_final):
    G, K, Cin = feat.shape
    cp1, cp2 = p['convs']
    C = cp2['W'].shape[1]
    args = [feat, dsc[..., None],
            cp1['W'], _row(cp1['b']), cp2['W'], _row(cp2['b']),
            p['wnet']['W'], _row(p['wnet']['b']),
            p['dnet1']['W'].reshape(1, 16), _row(p['dnet1']['b']),
            p['dnet2']['W'].reshape(1, 16), _row(p['dnet2']['b'])]
    in_specs = [pl.BlockSpec((gt, K, Cin), lambda i: (i, 0, 0)),
                pl.BlockSpec((gt, K, 1), lambda i: (i, 0, 0))]
    in_specs += [_full(a.shape) for a in args[2:]]
    if fuse_final:
        args += [p['final']['W'], _row(p['final']['b'])]
        in_specs += [_full(p['final']['W'].shape), _full((1, p['final']['W'].shape[1]))]
        cout = p['final']['W'].shape[1]
        body = _sa_mlp_final_kernel
    else:
        cout = C * 32
        body = _sa_mlp_nofinal_kernel
    return pl.pallas_call(
        body,
        grid=(G // gt,),
        in_specs=in_specs,
        out_specs=pl.BlockSpec((gt, cout), lambda i: (i, 0)),
        out_shape=jax.ShapeDtypeStruct((G, cout), jnp.float32),
    )(*args)


def _linear_relu_kernel(x_ref, w_ref, b_ref, o_ref):
    o_ref[...] = jnp.maximum(
        jnp.dot(x_ref[...], w_ref[...], preferred_element_type=jnp.float32)
        + b_ref[...], 0.0)


def _linear_relu(x, w, b, nt):
    M, Kin = x.shape
    N = w.shape[1]
    return pl.pallas_call(
        _linear_relu_kernel,
        grid=(N // nt,),
        in_specs=[pl.BlockSpec((M, Kin), lambda i: (0, 0)),
                  pl.BlockSpec((Kin, nt), lambda i: (0, i)),
                  pl.BlockSpec((1, nt), lambda i: (0, i))],
        out_specs=pl.BlockSpec((M, nt), lambda i: (0, i)),
        out_shape=jax.ShapeDtypeStruct((M, N), jnp.float32),
    )(x, w, b.reshape(1, -1))


def _head_kernel(x_ref, w1_ref, b1_ref, w2_ref, b2_ref, w3_ref, b3_ref,
                 o_ref):
    h = jnp.maximum(
        jnp.dot(x_ref[...], w1_ref[...], preferred_element_type=jnp.float32)
        + b1_ref[...], 0.0)
    h = jnp.maximum(
        jnp.dot(h, w2_ref[...], preferred_element_type=jnp.float32)
        + b2_ref[...], 0.0)
    l = jnp.dot(h, w3_ref[...], preferred_element_type=jnp.float32) + b3_ref[...]
    m = jnp.max(l, axis=-1, keepdims=True)
    e = jnp.exp(l - m)
    o_ref[...] = e / jnp.sum(e, axis=-1, keepdims=True)


def _head(x, d1, d2, d3):
    B = x.shape[0]
    nout = d3['W'].shape[1]
    args = [x, d1['W'], _row(d1['b']), d2['W'], _row(d2['b']),
            d3['W'], _row(d3['b'])]
    return pl.pallas_call(
        _head_kernel,
        in_specs=[_full(a.shape) for a in args],
        out_specs=_full((B, nout)),
        out_shape=jax.ShapeDtypeStruct((B, nout), jnp.float32),
    )(*args)


def _sa_stage(xyz, feature, p, npoint, K, radius, sigma, group_all, gt):
    B, N, _ = xyz.shape
    if feature is None:
        feature = xyz
    density = _kde_ball(xyz, radius, sigma, min(128, N))
    inv_d = 1.0 / density
    if group_all:
        new_xyz = jnp.zeros((B, 1, 3), xyz.dtype)
        feat = jnp.concatenate([xyz, feature], axis=-1)[:, None]
        gd = inv_d[:, None, :, :]
    else:
        fps_idx = _fps(xyz, npoint)
        new_xyz = _gather(xyz, fps_idx)
        idx = _knn(K, xyz, new_xyz)
        g_xyz = _gather(xyz, idx) - new_xyz[:, :, None, :]
        feat = jnp.concatenate([g_xyz, _gather(feature, idx)], axis=-1)
        gd = _gather(inv_d, idx)
    dmax = jnp.max(gd, axis=2, keepdims=True)
    dsc = (gd / dmax)[..., 0]
    M, Kg, Cin = feat.shape[1], feat.shape[2], feat.shape[3]
    G = B * M
    out = _sa_mlp(feat.reshape(G, Kg, Cin), dsc.reshape(G, Kg), p, gt,
                  fuse_final=not group_all)
    if group_all:
        out = _linear_relu(out, p['final']['W'], _row(p['final']['b'])[0],
                           256)
    return new_xyz, out.reshape(B, M, -1)


def kernel(points, params):
    xyz1, f1 = _sa_stage(points, None, params['sa1'], 512, 32, 0.1, 0.1,
                         False, 128)
    xyz2, f2 = _sa_stage(xyz1, f1, params['sa2'], 128, 32, 0.2, 0.2,
                         False, 128)
    _, f3 = _sa_stage(xyz2, f2, params['sa3'], 1, 32, 0.8, 0.4, True, 32)
    net = f3.reshape(points.shape[0], -1)
    return _head(net, params['d1'], params['d2'], params['d3'])
